# baseline (speedup 1.0000x reference)
import functools

import jax
import jax.numpy as jnp
from jax.experimental import pallas as pl
from jax.experimental.pallas import tpu as pltpu

H = 256
B = 1024
WIN_P, WIN_F = 5, 5
NWIN = WIN_P + WIN_F + 1


def _f32dot(x, w):
    return jax.lax.dot_general(x, w, (((x.ndim - 1,), (0,)), ((), ())),
                               preferred_element_type=jnp.float32)


def _ln(x, g, b):
    m = x.mean(-1, keepdims=True)
    v = ((x - m) ** 2).mean(-1, keepdims=True)
    return (x - m) * jax.lax.rsqrt(v + 1e-5) * g + b


def _relu(x):
    return jnp.maximum(x, 0.0)


def _pool(x, s):
    n, h, w, c = x.shape
    a = x.reshape(n * h // 2, 2, w, c).max(axis=1)
    at = jnp.swapaxes(a, -1, -2).reshape(n * h // 2 * c, w)
    r = _f32dot(at, s)
    m = jnp.maximum(r[:, :w // 2], r[:, w // 2:]).astype(jnp.bfloat16)
    m = jnp.swapaxes(m.reshape(n * h // 2, c, w // 2), -1, -2)
    return m.reshape(n, h // 2, w // 2, c)


def _conv_tap3(x, wk, b):
    n, h, w, cin = x.shape
    cout = wk.shape[-1]
    zr = jnp.zeros((n, 1, w, cin), x.dtype)
    xp = jnp.concatenate([zr, x, zr], axis=1)
    zc = jnp.zeros((n, h + 2, 1, cin), x.dtype)
    xp = jnp.concatenate([zc, xp, zc], axis=2)
    acc = jnp.broadcast_to(b, (n * h * w, cout))
    for ky in range(3):
        p = jnp.concatenate([xp[:, ky:ky + h, kx:kx + w, :] for kx in range(3)],
                            axis=-1)
        acc = acc + _f32dot(p.reshape(n * h * w, 3 * cin), wk[ky])
    return acc.reshape(n, h, w, cout)


def _face_body(x_ref, w1_ref, b1_ref, w2_ref, b2_ref, w3_ref, b3_ref,
               w4_ref, b4_ref, s1_ref, s2_ref, s3_ref, out_ref):
    nb = x_ref.shape[0]
    z_row9 = jnp.zeros((nb, 32, 9), jnp.bfloat16)
    z1 = jnp.zeros((nb, 1, 3), jnp.bfloat16)
    row9 = []
    for r in range(32):
        row = jnp.swapaxes(x_ref[:, :, r, :], 1, 2)
        left = jnp.concatenate([z1, row[:, :31, :]], axis=1)
        right = jnp.concatenate([row[:, 1:, :], z1], axis=1)
        row9.append(jnp.concatenate([left, row, right], axis=-1))
    patches = []
    for y in range(32):
        taps = [row9[y + k - 1] if 0 <= y + k - 1 <= 31 else z_row9
                for k in range(3)]
        patches.append(jnp.concatenate(taps, axis=-1))
    p1 = jnp.stack(patches, axis=1)
    h1 = _f32dot(p1.reshape(nb * 32 * 32, 27), w1_ref[...]) + b1_ref[...]
    h1 = _pool(_relu(h1.reshape(nb, 32, 32, 64)).astype(jnp.bfloat16), s1_ref[...])
    h2 = _pool(_relu(_conv_tap3(h1, w2_ref[...], b2_ref[...])).astype(jnp.bfloat16), s2_ref[...])
    h3 = _pool(_relu(_conv_tap3(h2, w3_ref[...], b3_ref[...])).astype(jnp.bfloat16), s3_ref[...])
    h4 = _relu(_conv_tap3(h3, w4_ref[...], b4_ref[...]))
    out_ref[...] = h4.reshape(nb, 16, 512).mean(axis=1)


def _const_spec(shape):
    n = len(shape)
    return pl.BlockSpec(shape, lambda i, _n=n: (0,) * _n)


def _face_cnn(fs_nhwc, wk, bk, nb):
    n_img = fs_nhwc.shape[0]
    grid = (n_img // nb,)
    in_specs = [pl.BlockSpec((nb, 3, 32, 32), lambda i: (i, 0, 0, 0))]
    operands = [fs_nhwc]
    for l in range(4):
        in_specs.append(_const_spec(wk[l].shape))
        in_specs.append(_const_spec(bk[l].shape))
        operands.extend([wk[l], bk[l]])
    for w in (32, 16, 8):
        x = jax.lax.broadcasted_iota(jnp.int32, (w, w), 0)
        j = jax.lax.broadcasted_iota(jnp.int32, (w, w), 1)
        sel = ((j < w // 2) & (x == 2 * j)) | ((j >= w // 2) & (x == 2 * (j - w // 2) + 1))
        s = sel.astype(jnp.bfloat16)
        in_specs.append(_const_spec(s.shape))
        operands.append(s)
    return pl.pallas_call(
        _face_body,
        grid=grid,
        in_specs=in_specs,
        out_specs=pl.BlockSpec((nb, 512), lambda i: (i, 0)),
        out_shape=jax.ShapeDtypeStruct((n_img, 512), jnp.float32),
    )(*operands)


def _enc_body(t0_ref, t1_ref, t2_ref, t3_ref, wt_ref, bt_ref, tg_ref, tb_ref,
              wf_ref, bf_ref,
              au_ref, am_ref, wa1_ref, ba1_ref, wa2_ref, ba2_ref, ag_ref, ab_ref,
              ff_ref, wcls_ref, bcls_ref, wenc_ref, benc_ref, vg_ref, vb_ref,
              c0_ref, c1_ref, c2_ref, c3_ref, c4_ref, c5_ref, c6_ref, c7_ref,
              c8_ref, wc_ref, bc_ref, cg_ref, cb_ref,
              wat1_ref, bat1_ref, wat2_ref, bat2_ref,
              t_out, a_out, v_out, c_out):
    tenc = []
    trefs = [t0_ref, t1_ref, t2_ref, t3_ref]
    for i in range(4):
        x = _f32dot(trefs[i][...], wt_ref[i]) + bt_ref[i:i + 1]
        tenc.append(_relu(_ln(x, tg_ref[i:i + 1], tb_ref[i:i + 1])))
    tcat = jnp.concatenate(tenc, axis=-1)
    t_out[...] = _relu(_f32dot(tcat, wf_ref[...]) + bf_ref[...])
    au = au_ref[...]
    m = am_ref[...]
    msum = (au * m[:, None, :]).sum(axis=-1)
    lens = m.sum(axis=-1, keepdims=True)
    apool = msum / (lens + 1e-8)
    a1 = _relu(_f32dot(apool, wa1_ref[...]) + ba1_ref[...])
    a2 = _f32dot(a1, wa2_ref[...]) + ba2_ref[...]
    a_out[...] = _relu(_ln(a2, ag_ref[...], ab_ref[...]))
    fcls = _relu(_f32dot(ff_ref[...], wcls_ref[...]) + bcls_ref[...])
    r = fcls.shape[0] // 2
    fmean = fcls.reshape(r, 2, 512).mean(axis=1)
    fenc = _f32dot(fmean, wenc_ref[...]) + benc_ref[...]
    v_out[...] = _relu(_ln(fenc, vg_ref[...], vb_ref[...]))
    crefs = [c0_ref, c1_ref, c2_ref, c3_ref, c4_ref, c5_ref, c6_ref, c7_ref,
             c8_ref]
    cenc, scores = [], []
    for i in range(9):
        x = _f32dot(crefs[i][...], wc_ref[i]) + bc_ref[i:i + 1]
        e = _relu(_ln(x, cg_ref[i:i + 1], cb_ref[i:i + 1]))
        cenc.append(e)
        th = jnp.tanh(_f32dot(e, wat1_ref[...]) + bat1_ref[...])
        scores.append((th * wat2_ref[...]).sum(axis=-1, keepdims=True)
                      + bat2_ref[...])
    s = jnp.concatenate(scores, axis=-1)
    s = s - s.max(axis=-1, keepdims=True)
    e = jnp.exp(s)
    aw = e / e.sum(axis=-1, keepdims=True)
    acc = aw[:, 0:1] * cenc[0]
    for i in range(1, 9):
        acc = acc + aw[:, i:i + 1] * cenc[i]
    c_out[...] = acc


def _encoders(inputs, params, ff, r):
    grid = (B // r,)
    rowspec = lambda shape: pl.BlockSpec((r,) + shape[1:],
                                         lambda i: (i,) + (0,) * (len(shape) - 1))
    p = params
    stack = lambda seq: jnp.stack(seq, axis=0)
    row2 = lambda v: v.reshape(1, -1)

    wt = stack([p["text_layers"][i]["lin"]["W"] for i in range(4)])
    bt = stack([p["text_layers"][i]["lin"]["b"] for i in range(4)])
    tg = stack([p["text_layers"][i]["ln"]["g"] for i in range(4)])
    tb = stack([p["text_layers"][i]["ln"]["b"] for i in range(4)])
    wc = stack([p["cs_layers"][i]["lin"]["W"] for i in range(9)])
    bc = stack([p["cs_layers"][i]["lin"]["b"] for i in range(9)])
    cg = stack([p["cs_layers"][i]["ln"]["g"] for i in range(9)])
    cb = stack([p["cs_layers"][i]["ln"]["b"] for i in range(9)])
    au_t = jnp.transpose(inputs["audio"], (0, 2, 1))

    operands = []
    in_specs = []

    def add(x, spec):
        operands.append(x)
        in_specs.append(spec)

    for i in range(4):
        add(inputs[f"text_{i}"], rowspec((B, 1024)))
    add(wt, _const_spec(wt.shape))
    add(bt, _const_spec(bt.shape))
    add(tg, _const_spec(tg.shape))
    add(tb, _const_spec(tb.shape))
    add(p["text_fusion"]["W"], _const_spec((1024, H)))
    add(row2(p["text_fusion"]["b"]), _const_spec((1, H)))
    add(au_t, rowspec((B, 13, 200)))
    add(inputs["audio_mask"], rowspec((B, 200)))
    add(p["audio_enc1"]["W"], _const_spec((13, 128)))
    add(row2(p["audio_enc1"]["b"]), _const_spec((1, 128)))
    add(p["audio_enc2"]["W"], _const_spec((128, H)))
    add(row2(p["audio_enc2"]["b"]), _const_spec((1, H)))
    add(row2(p["audio_ln"]["g"]), _const_spec((1, H)))
    add(row2(p["audio_ln"]["b"]), _const_spec((1, H)))
    add(ff, pl.BlockSpec((2 * r, 512), lambda i: (i, 0)))
    add(p["face_cls"]["W"], _const_spec((512, 512)))
    add(row2(p["face_cls"]["b"]), _const_spec((1, 512)))
    add(p["face_enc"]["lin"]["W"], _const_spec((512, H)))
    add(row2(p["face_enc"]["lin"]["b"]), _const_spec((1, H)))
    add(row2(p["face_enc"]["ln"]["g"]), _const_spec((1, H)))
    add(row2(p["face_enc"]["ln"]["b"]), _const_spec((1, H)))
    for i in range(9):
        add(inputs[f"cs_{i}"], rowspec((B, 768)))
    add(wc, _const_spec(wc.shape))
    add(bc, _const_spec(bc.shape))
    add(cg, _const_spec(cg.shape))
    add(cb, _const_spec(cb.shape))
    add(p["cs_attn1"]["W"], _const_spec((H, 128)))
    add(row2(p["cs_attn1"]["b"]), _const_spec((1, 128)))
    add(p["cs_attn2"]["W"].reshape(1, 128), _const_spec((1, 128)))
    add(p["cs_attn2"]["b"].reshape(1, 1), _const_spec((1, 1)))

    out_shape = [jax.ShapeDtypeStruct((B, H), jnp.float32)] * 4
    out_specs = [pl.BlockSpec((r, H), lambda i: (i, 0))] * 4
    return pl.pallas_call(
        _enc_body,
        grid=grid,
        in_specs=in_specs,
        out_specs=out_specs,
        out_shape=out_shape,
    )(*operands)


def _wmean(x):
    top = jnp.broadcast_to(x[0:1], (WIN_P, H))
    bot = jnp.broadcast_to(x[B - 1:B], (WIN_F, H))
    xp = jnp.concatenate([top, x, bot], axis=0)
    s = xp[0:B]
    for k in range(1, NWIN):
        s = s + xp[k:k + B]
    return s * (1.0 / NWIN)


def _heter_body(x1_ref, x2_ref, w1_ref, b1_ref, w2_ref, b2_ref, out_ref):
    x1 = x1_ref[0]
    x2 = x2_ref[0]
    w1 = w1_ref[0]
    w2 = w2_ref[0]
    b1 = b1_ref[0:1, 0]
    b2 = b2_ref[0:1, 0]
    m1 = _wmean(x1)
    m2 = _wmean(x2)
    n1 = _relu(_f32dot(x1, w1[:H]) + _f32dot(m2, w1[H:]) + b1)
    n2 = _relu(_f32dot(x2, w1[:H]) + _f32dot(m1, w1[H:]) + b1)
    y1 = x1 + n1
    y2 = x2 + n2
    a1 = _wmean(y1)
    a2 = _wmean(y2)
    n1 = _relu(_f32dot(y1, w2[:H]) + _f32dot(a2, w2[H:]) + b2)
    n2 = _relu(_f32dot(y2, w2[:H]) + _f32dot(a1, w2[H:]) + b2)
    out_ref[0] = y1 + n1 + y2 + n2


def _heter(t, a, v, c, params):
    hp = params["heter"]
    keys = ["ta", "tv", "av", "tc", "ac", "vc"]
    src = {"t": t, "a": a, "v": v, "c": c}
    x1 = jnp.stack([src[k[0]] for k in keys], axis=0)
    x2 = jnp.stack([src[k[1]] for k in keys], axis=0)
    w1 = jnp.stack([hp[k][0]["W"] for k in keys], axis=0)
    b1 = jnp.stack([hp[k][0]["b"].reshape(1, H) for k in keys], axis=0)
    w2 = jnp.stack([hp[k][1]["W"] for k in keys], axis=0)
    b2 = jnp.stack([hp[k][1]["b"].reshape(1, H) for k in keys], axis=0)
    return pl.pallas_call(
        _heter_body,
        grid=(6,),
        in_specs=[
            pl.BlockSpec((1, B, H), lambda i: (i, 0, 0)),
            pl.BlockSpec((1, B, H), lambda i: (i, 0, 0)),
            pl.BlockSpec((1, 2 * H, H), lambda i: (i, 0, 0)),
            pl.BlockSpec((1, 1, H), lambda i: (i, 0, 0)),
            pl.BlockSpec((1, 2 * H, H), lambda i: (i, 0, 0)),
            pl.BlockSpec((1, 1, H), lambda i: (i, 0, 0)),
        ],
        out_specs=pl.BlockSpec((1, B, H), lambda i: (i, 0, 0)),
        out_shape=jax.ShapeDtypeStruct((6, B, H), jnp.float32),
    )(x1, x2, w1, b1, w2, b2)


def _fusion_body(f_ref, wm_ref, bm_ref, mg_ref, mb_ref, wh_ref, bh_ref,
                 out_ref):
    wm = wm_ref[...]
    acc = jnp.broadcast_to(bm_ref[...], (f_ref.shape[1], H))
    for p in range(6):
        acc = acc + _f32dot(f_ref[p], wm[p * H:(p + 1) * H])
    fused = _relu(_ln(acc, mg_ref[...], mb_ref[...]))
    out_ref[...] = _f32dot(fused, wh_ref[...]) + bh_ref[...]


def _fusion(feats, params, r):
    p = params
    row2 = lambda v: v.reshape(1, -1)
    wh = jnp.concatenate([p["emo"]["W"], p["sent"]["W"]], axis=-1)
    bh = jnp.concatenate([p["emo"]["b"], p["sent"]["b"]]).reshape(1, 10)
    return pl.pallas_call(
        _fusion_body,
        grid=(B // r,),
        in_specs=[
            pl.BlockSpec((6, r, H), lambda i: (0, i, 0)),
            _const_spec((6 * H, H)),
            _const_spec((1, H)),
            _const_spec((1, H)),
            _const_spec((1, H)),
            _const_spec((H, 10)),
            _const_spec((1, 10)),
        ],
        out_specs=pl.BlockSpec((r, 10), lambda i: (i, 0)),
        out_shape=jax.ShapeDtypeStruct((B, 10), jnp.float32),
    )(feats, p["modal_fusion"]["lin"]["W"], row2(p["modal_fusion"]["lin"]["b"]),
      row2(p["modal_fusion"]["ln"]["g"]), row2(p["modal_fusion"]["ln"]["b"]),
      wh, bh)


def _prep_conv_weights(params):
    wk, bk = [], []
    for l, cw in enumerate(params["face_convs"]):
        w = cw["W"]
        cout, cin = w.shape[0], w.shape[1]
        wt = jnp.transpose(w, (2, 3, 1, 0)).astype(jnp.bfloat16)
        if l == 0:
            wk.append(wt.reshape(27, cout))
        else:
            wk.append(wt.reshape(3, 3 * cin, cout))
        bk.append(cw["b"].reshape(1, cout))
    return wk, bk


def kernel(inputs, params):
    fs = inputs["face_sequences"]
    n_img = fs.shape[0] * fs.shape[1]
    fs_nhwc = fs.reshape(n_img, 3, 32, 32).astype(jnp.bfloat16)
    wk, bk = _prep_conv_weights(params)
    ff = _face_cnn(fs_nhwc, wk, bk, nb=32)
    t, a, v, c = _encoders(inputs, params, ff, r=128)
    feats = _heter(t, a, v, c, params)
    return _fusion(feats, params, r=256)


# speedup vs baseline: 1.1880x; 1.1880x over previous
import functools

import jax
import jax.numpy as jnp
from jax.experimental import pallas as pl
from jax.experimental.pallas import tpu as pltpu

H = 256
B = 1024
WIN_P, WIN_F = 5, 5
NWIN = WIN_P + WIN_F + 1


def _f32dot(x, w):
    return jax.lax.dot_general(x, w, (((x.ndim - 1,), (0,)), ((), ())),
                               preferred_element_type=jnp.float32)


def _ln(x, g, b):
    m = x.mean(-1, keepdims=True)
    v = ((x - m) ** 2).mean(-1, keepdims=True)
    return (x - m) * jax.lax.rsqrt(v + 1e-5) * g + b


def _relu(x):
    return jnp.maximum(x, 0.0)


def _pool(x, s):
    n, h, w, c = x.shape
    a = x.reshape(n * h // 2, 2, w, c).max(axis=1)
    at = jnp.swapaxes(a, -1, -2).reshape(n * h // 2 * c, w)
    r = _f32dot(at, s)
    m = jnp.maximum(r[:, :w // 2], r[:, w // 2:]).astype(jnp.bfloat16)
    m = jnp.swapaxes(m.reshape(n * h // 2, c, w // 2), -1, -2)
    return m.reshape(n, h // 2, w // 2, c)


def _conv_tap3(x, wk, b):
    n, h, w, cin = x.shape
    cout = wk.shape[-1]
    zr = jnp.zeros((n, 1, w, cin), x.dtype)
    xp = jnp.concatenate([zr, x, zr], axis=1)
    zc = jnp.zeros((n, h + 2, 1, cin), x.dtype)
    xp = jnp.concatenate([zc, xp, zc], axis=2)
    acc = jnp.broadcast_to(b, (n * h * w, cout))
    for ky in range(3):
        p = jnp.concatenate([xp[:, ky:ky + h, kx:kx + w, :] for kx in range(3)],
                            axis=-1)
        acc = acc + _f32dot(p.reshape(n * h * w, 3 * cin), wk[ky])
    return acc.reshape(n, h, w, cout)


def _face_body(x_ref, w1_ref, b1_ref, w2_ref, b2_ref, w3_ref, b3_ref,
               w4_ref, b4_ref, s1_ref, s2_ref, s3_ref, out_ref):
    nb = x_ref.shape[0]
    z_row9 = jnp.zeros((nb, 32, 9), jnp.bfloat16)
    z1 = jnp.zeros((nb, 1, 3), jnp.bfloat16)
    row9 = []
    for r in range(32):
        row = jnp.swapaxes(x_ref[:, :, r, :], 1, 2)
        left = jnp.concatenate([z1, row[:, :31, :]], axis=1)
        right = jnp.concatenate([row[:, 1:, :], z1], axis=1)
        row9.append(jnp.concatenate([left, row, right], axis=-1))
    patches = []
    for y in range(32):
        taps = [row9[y + k - 1] if 0 <= y + k - 1 <= 31 else z_row9
                for k in range(3)]
        patches.append(jnp.concatenate(taps, axis=-1))
    p1 = jnp.stack(patches, axis=1)
    h1 = _f32dot(p1.reshape(nb * 32 * 32, 27), w1_ref[...]) + b1_ref[...]
    h1 = _pool(_relu(h1.reshape(nb, 32, 32, 64)).astype(jnp.bfloat16), s1_ref[...])
    h2 = _pool(_relu(_conv_tap3(h1, w2_ref[...], b2_ref[...])).astype(jnp.bfloat16), s2_ref[...])
    h3 = _pool(_relu(_conv_tap3(h2, w3_ref[...], b3_ref[...])).astype(jnp.bfloat16), s3_ref[...])
    h4 = _relu(_conv_tap3(h3, w4_ref[...], b4_ref[...]))
    out_ref[...] = h4.reshape(nb, 16, 512).mean(axis=1)


def _const_spec(shape):
    n = len(shape)
    return pl.BlockSpec(shape, lambda i, _n=n: (0,) * _n)


def _face_cnn(fs_nhwc, wk, bk, nb):
    n_img = fs_nhwc.shape[0]
    grid = (n_img // nb,)
    in_specs = [pl.BlockSpec((nb, 3, 32, 32), lambda i: (i, 0, 0, 0))]
    operands = [fs_nhwc]
    for l in range(4):
        in_specs.append(_const_spec(wk[l].shape))
        in_specs.append(_const_spec(bk[l].shape))
        operands.extend([wk[l], bk[l]])
    for w in (32, 16, 8):
        x = jax.lax.broadcasted_iota(jnp.int32, (w, w), 0)
        j = jax.lax.broadcasted_iota(jnp.int32, (w, w), 1)
        sel = ((j < w // 2) & (x == 2 * j)) | ((j >= w // 2) & (x == 2 * (j - w // 2) + 1))
        s = sel.astype(jnp.bfloat16)
        in_specs.append(_const_spec(s.shape))
        operands.append(s)
    return pl.pallas_call(
        _face_body,
        grid=grid,
        in_specs=in_specs,
        out_specs=pl.BlockSpec((nb, 512), lambda i: (i, 0)),
        out_shape=jax.ShapeDtypeStruct((n_img, 512), jnp.float32),
    )(*operands)


def _enc_body(t0_ref, t1_ref, t2_ref, t3_ref, wt_ref, bt_ref, tg_ref, tb_ref,
              wf_ref, bf_ref,
              au_ref, am_ref, wa1_ref, ba1_ref, wa2_ref, ba2_ref, ag_ref, ab_ref,
              ff_ref, wcls_ref, bcls_ref, wenc_ref, benc_ref, vg_ref, vb_ref,
              c0_ref, c1_ref, c2_ref, c3_ref, c4_ref, c5_ref, c6_ref, c7_ref,
              c8_ref, wc_ref, bc_ref, cg_ref, cb_ref,
              wat1_ref, bat1_ref, wat2_ref, bat2_ref,
              t_out, a_out, v_out, c_out):
    tenc = []
    trefs = [t0_ref, t1_ref, t2_ref, t3_ref]
    for i in range(4):
        x = _f32dot(trefs[i][...], wt_ref[i]) + bt_ref[i:i + 1]
        tenc.append(_relu(_ln(x, tg_ref[i:i + 1], tb_ref[i:i + 1])))
    tcat = jnp.concatenate(tenc, axis=-1)
    t_out[...] = _relu(_f32dot(tcat, wf_ref[...]) + bf_ref[...])
    au = au_ref[...]
    m = am_ref[...]
    msum = (au * m[:, None, :]).sum(axis=-1)
    lens = m.sum(axis=-1, keepdims=True)
    apool = msum / (lens + 1e-8)
    a1 = _relu(_f32dot(apool, wa1_ref[...]) + ba1_ref[...])
    a2 = _f32dot(a1, wa2_ref[...]) + ba2_ref[...]
    a_out[...] = _relu(_ln(a2, ag_ref[...], ab_ref[...]))
    fcls = _relu(_f32dot(ff_ref[...], wcls_ref[...]) + bcls_ref[...])
    r = fcls.shape[0] // 2
    fmean = fcls.reshape(r, 2, 512).mean(axis=1)
    fenc = _f32dot(fmean, wenc_ref[...]) + benc_ref[...]
    v_out[...] = _relu(_ln(fenc, vg_ref[...], vb_ref[...]))
    crefs = [c0_ref, c1_ref, c2_ref, c3_ref, c4_ref, c5_ref, c6_ref, c7_ref,
             c8_ref]
    cenc, scores = [], []
    for i in range(9):
        x = _f32dot(crefs[i][...], wc_ref[i]) + bc_ref[i:i + 1]
        e = _relu(_ln(x, cg_ref[i:i + 1], cb_ref[i:i + 1]))
        cenc.append(e)
        th = jnp.tanh(_f32dot(e, wat1_ref[...]) + bat1_ref[...])
        scores.append((th * wat2_ref[...]).sum(axis=-1, keepdims=True)
                      + bat2_ref[...])
    s = jnp.concatenate(scores, axis=-1)
    s = s - s.max(axis=-1, keepdims=True)
    e = jnp.exp(s)
    aw = e / e.sum(axis=-1, keepdims=True)
    acc = aw[:, 0:1] * cenc[0]
    for i in range(1, 9):
        acc = acc + aw[:, i:i + 1] * cenc[i]
    c_out[...] = acc


def _encoders(inputs, params, ff, r):
    grid = (B // r,)
    rowspec = lambda shape: pl.BlockSpec((r,) + shape[1:],
                                         lambda i: (i,) + (0,) * (len(shape) - 1))
    p = params
    stack = lambda seq: jnp.stack(seq, axis=0)
    row2 = lambda v: v.reshape(1, -1)

    wt = stack([p["text_layers"][i]["lin"]["W"] for i in range(4)])
    bt = stack([p["text_layers"][i]["lin"]["b"] for i in range(4)])
    tg = stack([p["text_layers"][i]["ln"]["g"] for i in range(4)])
    tb = stack([p["text_layers"][i]["ln"]["b"] for i in range(4)])
    wc = stack([p["cs_layers"][i]["lin"]["W"] for i in range(9)])
    bc = stack([p["cs_layers"][i]["lin"]["b"] for i in range(9)])
    cg = stack([p["cs_layers"][i]["ln"]["g"] for i in range(9)])
    cb = stack([p["cs_layers"][i]["ln"]["b"] for i in range(9)])
    au_t = jnp.transpose(inputs["audio"], (0, 2, 1))

    operands = []
    in_specs = []

    def add(x, spec):
        operands.append(x)
        in_specs.append(spec)

    for i in range(4):
        add(inputs[f"text_{i}"], rowspec((B, 1024)))
    add(wt, _const_spec(wt.shape))
    add(bt, _const_spec(bt.shape))
    add(tg, _const_spec(tg.shape))
    add(tb, _const_spec(tb.shape))
    add(p["text_fusion"]["W"], _const_spec((1024, H)))
    add(row2(p["text_fusion"]["b"]), _const_spec((1, H)))
    add(au_t, rowspec((B, 13, 200)))
    add(inputs["audio_mask"], rowspec((B, 200)))
    add(p["audio_enc1"]["W"], _const_spec((13, 128)))
    add(row2(p["audio_enc1"]["b"]), _const_spec((1, 128)))
    add(p["audio_enc2"]["W"], _const_spec((128, H)))
    add(row2(p["audio_enc2"]["b"]), _const_spec((1, H)))
    add(row2(p["audio_ln"]["g"]), _const_spec((1, H)))
    add(row2(p["audio_ln"]["b"]), _const_spec((1, H)))
    add(ff, pl.BlockSpec((2 * r, 512), lambda i: (i, 0)))
    add(p["face_cls"]["W"], _const_spec((512, 512)))
    add(row2(p["face_cls"]["b"]), _const_spec((1, 512)))
    add(p["face_enc"]["lin"]["W"], _const_spec((512, H)))
    add(row2(p["face_enc"]["lin"]["b"]), _const_spec((1, H)))
    add(row2(p["face_enc"]["ln"]["g"]), _const_spec((1, H)))
    add(row2(p["face_enc"]["ln"]["b"]), _const_spec((1, H)))
    for i in range(9):
        add(inputs[f"cs_{i}"], rowspec((B, 768)))
    add(wc, _const_spec(wc.shape))
    add(bc, _const_spec(bc.shape))
    add(cg, _const_spec(cg.shape))
    add(cb, _const_spec(cb.shape))
    add(p["cs_attn1"]["W"], _const_spec((H, 128)))
    add(row2(p["cs_attn1"]["b"]), _const_spec((1, 128)))
    add(p["cs_attn2"]["W"].reshape(1, 128), _const_spec((1, 128)))
    add(p["cs_attn2"]["b"].reshape(1, 1), _const_spec((1, 1)))

    out_shape = [jax.ShapeDtypeStruct((B, H), jnp.float32)] * 4
    out_specs = [pl.BlockSpec((r, H), lambda i: (i, 0))] * 4
    return pl.pallas_call(
        _enc_body,
        grid=grid,
        in_specs=in_specs,
        out_specs=out_specs,
        out_shape=out_shape,
    )(*operands)


def _wmean(x):
    top = jnp.broadcast_to(x[0:1], (WIN_P, H))
    bot = jnp.broadcast_to(x[B - 1:B], (WIN_F, H))
    xp = jnp.concatenate([top, x, bot], axis=0)
    s = xp[0:B]
    for k in range(1, NWIN):
        s = s + xp[k:k + B]
    return s * (1.0 / NWIN)


def _heter_body(x1_ref, x2_ref, w1_ref, b1_ref, w2_ref, b2_ref, out_ref):
    x1 = x1_ref[0]
    x2 = x2_ref[0]
    w1 = w1_ref[0]
    w2 = w2_ref[0]
    b1 = b1_ref[0:1, 0]
    b2 = b2_ref[0:1, 0]
    m1 = _wmean(x1)
    m2 = _wmean(x2)
    n1 = _relu(_f32dot(x1, w1[:H]) + _f32dot(m2, w1[H:]) + b1)
    n2 = _relu(_f32dot(x2, w1[:H]) + _f32dot(m1, w1[H:]) + b1)
    y1 = x1 + n1
    y2 = x2 + n2
    a1 = _wmean(y1)
    a2 = _wmean(y2)
    n1 = _relu(_f32dot(y1, w2[:H]) + _f32dot(a2, w2[H:]) + b2)
    n2 = _relu(_f32dot(y2, w2[:H]) + _f32dot(a1, w2[H:]) + b2)
    out_ref[0] = y1 + n1 + y2 + n2


def _heter(t, a, v, c, params):
    hp = params["heter"]
    keys = ["ta", "tv", "av", "tc", "ac", "vc"]
    src = {"t": t, "a": a, "v": v, "c": c}
    x1 = jnp.stack([src[k[0]] for k in keys], axis=0)
    x2 = jnp.stack([src[k[1]] for k in keys], axis=0)
    w1 = jnp.stack([hp[k][0]["W"] for k in keys], axis=0)
    b1 = jnp.stack([hp[k][0]["b"].reshape(1, H) for k in keys], axis=0)
    w2 = jnp.stack([hp[k][1]["W"] for k in keys], axis=0)
    b2 = jnp.stack([hp[k][1]["b"].reshape(1, H) for k in keys], axis=0)
    return pl.pallas_call(
        _heter_body,
        grid=(6,),
        in_specs=[
            pl.BlockSpec((1, B, H), lambda i: (i, 0, 0)),
            pl.BlockSpec((1, B, H), lambda i: (i, 0, 0)),
            pl.BlockSpec((1, 2 * H, H), lambda i: (i, 0, 0)),
            pl.BlockSpec((1, 1, H), lambda i: (i, 0, 0)),
            pl.BlockSpec((1, 2 * H, H), lambda i: (i, 0, 0)),
            pl.BlockSpec((1, 1, H), lambda i: (i, 0, 0)),
        ],
        out_specs=pl.BlockSpec((1, B, H), lambda i: (i, 0, 0)),
        out_shape=jax.ShapeDtypeStruct((6, B, H), jnp.float32),
    )(x1, x2, w1, b1, w2, b2)


def _fusion_body(f_ref, wm_ref, bm_ref, mg_ref, mb_ref, wh_ref, bh_ref,
                 out_ref):
    wm = wm_ref[...]
    acc = jnp.broadcast_to(bm_ref[...], (f_ref.shape[1], H))
    for p in range(6):
        acc = acc + _f32dot(f_ref[p], wm[p * H:(p + 1) * H])
    fused = _relu(_ln(acc, mg_ref[...], mb_ref[...]))
    out_ref[...] = _f32dot(fused, wh_ref[...]) + bh_ref[...]


def _fusion(feats, params, r):
    p = params
    row2 = lambda v: v.reshape(1, -1)
    wh = jnp.concatenate([p["emo"]["W"], p["sent"]["W"]], axis=-1)
    bh = jnp.concatenate([p["emo"]["b"], p["sent"]["b"]]).reshape(1, 10)
    return pl.pallas_call(
        _fusion_body,
        grid=(B // r,),
        in_specs=[
            pl.BlockSpec((6, r, H), lambda i: (0, i, 0)),
            _const_spec((6 * H, H)),
            _const_spec((1, H)),
            _const_spec((1, H)),
            _const_spec((1, H)),
            _const_spec((H, 10)),
            _const_spec((1, 10)),
        ],
        out_specs=pl.BlockSpec((r, 10), lambda i: (i, 0)),
        out_shape=jax.ShapeDtypeStruct((B, 10), jnp.float32),
    )(feats, p["modal_fusion"]["lin"]["W"], row2(p["modal_fusion"]["lin"]["b"]),
      row2(p["modal_fusion"]["ln"]["g"]), row2(p["modal_fusion"]["ln"]["b"]),
      wh, bh)


def _prep_conv_weights(params):
    wk, bk = [], []
    for l, cw in enumerate(params["face_convs"]):
        w = cw["W"]
        cout, cin = w.shape[0], w.shape[1]
        wt = jnp.transpose(w, (2, 3, 1, 0)).astype(jnp.bfloat16)
        if l == 0:
            wk.append(wt.reshape(27, cout))
        else:
            wk.append(wt.reshape(3, 3 * cin, cout))
        bk.append(cw["b"].reshape(1, cout))
    return wk, bk


def kernel(inputs, params):
    fs = inputs["face_sequences"]
    n_img = fs.shape[0] * fs.shape[1]
    fs_nhwc = fs.reshape(n_img, 3, 32, 32).astype(jnp.bfloat16)
    wk, bk = _prep_conv_weights(params)
    ff = _face_cnn(fs_nhwc, wk, bk, nb=16)
    t, a, v, c = _encoders(inputs, params, ff, r=128)
    feats = _heter(t, a, v, c, params)
    return _fusion(feats, params, r=256)
